# baseline (speedup 1.0000x reference)
import functools

import jax
import jax.numpy as jnp
from jax import lax
from jax.experimental import pallas as pl
from jax.experimental.pallas import tpu as pltpu
from jax.experimental.pallas import tpu_sc as plsc

_D = 64
_SCALE = 8.0
_LANES = 16
_CHUNK = 128


@functools.lru_cache(maxsize=None)
def _make(vocab, batch, num_cores, num_subcores):
    num_workers = num_cores * num_subcores
    assert batch % (num_workers * _CHUNK) == 0
    b_per_w = batch // num_workers
    n_chunks = b_per_w // _CHUNK
    mesh = plsc.VectorSubcoreMesh(core_axis_name="c", subcore_axis_name="s")

    @functools.partial(
        pl.kernel,
        mesh=mesh,
        out_type=jax.ShapeDtypeStruct((batch, _D), jnp.float32),
        scratch_types=[
            pltpu.VMEM((b_per_w,), jnp.int32),
            pltpu.VMEM((_CHUNK, _D), jnp.float32),
            pltpu.SemaphoreType.DMA,
        ],
        compiler_params=pltpu.CompilerParams(use_tc_tiling_on_sc=False),
    )
    def k(idx_hbm, table_hbm, out_hbm, idx_v, rows_v, sem):
        wid = lax.axis_index("s") * num_cores + lax.axis_index("c")
        base = wid * b_per_w
        pltpu.sync_copy(idx_hbm.at[pl.ds(base, b_per_w)], idx_v)

        def chunk(j, carry):
            pltpu.async_copy(
                table_hbm.at[idx_v.at[pl.ds(j * _CHUNK, _CHUNK)]], rows_v, sem
            ).wait()

            def scale(r, c2):
                for q in range(_D // _LANES):
                    sl = pl.ds(q * _LANES, _LANES)
                    rows_v[r, sl] = rows_v[r, sl] * _SCALE
                return c2

            lax.fori_loop(0, _CHUNK, scale, None)
            pltpu.sync_copy(rows_v, out_hbm.at[pl.ds(base + j * _CHUNK, _CHUNK)])
            return carry

        lax.fori_loop(0, n_chunks, chunk, None)

    return k


def kernel(x, embedding):
    b0, s = x.shape
    batch = b0 * s
    xf = x.reshape(batch)
    info = plsc.get_sparse_core_info()
    out = _make(embedding.shape[0], batch, info.num_cores, info.num_subcores)(
        xf, embedding
    )
    return out.reshape(b0, s, _D)


# speedup vs baseline: 1.0747x; 1.0747x over previous
import functools

import jax
import jax.numpy as jnp
from jax import lax
from jax.experimental import pallas as pl
from jax.experimental.pallas import tpu as pltpu
from jax.experimental.pallas import tpu_sc as plsc

_D = 64
_SCALE = 8.0
_LANES = 16
_CHUNK = 128
_NBUF = 5


@functools.lru_cache(maxsize=None)
def _make(vocab, batch, num_cores, num_subcores):
    num_workers = num_cores * num_subcores
    group = _CHUNK * _NBUF
    assert batch % (num_workers * 2 * group) == 0
    b_per_w = batch // num_workers
    n_groups = b_per_w // group
    mesh = plsc.VectorSubcoreMesh(core_axis_name="c", subcore_axis_name="s")

    @functools.partial(
        pl.kernel,
        mesh=mesh,
        out_type=jax.ShapeDtypeStruct((batch, _D), jnp.float32),
        scratch_types=[
            pltpu.VMEM((b_per_w,), jnp.int32),
            pltpu.VMEM((2, _NBUF, _CHUNK, _D), jnp.float32),
            pltpu.SemaphoreType.DMA((2,)),
            pltpu.SemaphoreType.DMA((2,)),
        ],
        compiler_params=pltpu.CompilerParams(use_tc_tiling_on_sc=False),
    )
    def k(idx_hbm, table_hbm, out_hbm, idx_v, rows_v, gsem, wsem):
        wid = lax.axis_index("s") * num_cores + lax.axis_index("c")
        base = wid * b_per_w
        pltpu.sync_copy(idx_hbm.at[pl.ds(base, b_per_w)], idx_v)

        def fire_gathers(t, si):
            for b in range(_NBUF):
                off = (t * _NBUF + b) * _CHUNK
                pltpu.async_copy(
                    table_hbm.at[idx_v.at[pl.ds(off, _CHUNK)]],
                    rows_v.at[si, b],
                    gsem.at[si],
                )

        def drain(sem_arr, si, hbm_side):
            for b in range(_NBUF):
                pltpu.make_async_copy(hbm_side, rows_v.at[si, b], sem_arr.at[si]).wait()

        fire_gathers(0, 0)
        for t in range(n_groups):
            si = t % 2
            ni = 1 - si
            if t + 1 < n_groups:
                if t >= 1:
                    drain(wsem, ni, out_hbm.at[pl.ds(base, _CHUNK)])
                fire_gathers(t + 1, ni)
            drain(gsem, si, table_hbm.at[idx_v.at[pl.ds(0, _CHUNK)]])

            def scale_row(r, carry):
                for b in range(_NBUF):
                    for q in range(_D // _LANES):
                        sl = pl.ds(q * _LANES, _LANES)
                        rows_v[si, b, r, sl] = rows_v[si, b, r, sl] * _SCALE
                return carry

            lax.fori_loop(0, _CHUNK, scale_row, None)

            for b in range(_NBUF):
                off = (t * _NBUF + b) * _CHUNK
                pltpu.async_copy(
                    rows_v.at[si, b],
                    out_hbm.at[pl.ds(base + off, _CHUNK)],
                    wsem.at[si],
                )
        drain(wsem, (n_groups - 1) % 2, out_hbm.at[pl.ds(base, _CHUNK)])

    return k


def kernel(x, embedding):
    b0, s = x.shape
    batch = b0 * s
    xf = x.reshape(batch)
    info = plsc.get_sparse_core_info()
    out = _make(embedding.shape[0], batch, info.num_cores, info.num_subcores)(
        xf, embedding
    )
    return out.reshape(b0, s, _D)
